# baseline (speedup 1.0000x reference)
import functools

import jax
import jax.numpy as jnp
from jax import lax
from jax.experimental import pallas as pl
from jax.experimental.pallas import tpu as pltpu
from jax.experimental.pallas import tpu_sc as plsc

_V = 1000000
_D = 64
_H = 256
_OUT = 128
_B = 16384
_N = 2 * _B

_NC = 2
_NS = 16
_NW = _NC * _NS
_B_PER_W = _N // _NW
_CHUNK = 256
_NCHUNK = _B_PER_W // _CHUNK


def _sc_gather_lines(emb_lines, line_idx):
    mesh = plsc.VectorSubcoreMesh(core_axis_name="c", subcore_axis_name="s")

    @functools.partial(
        pl.kernel,
        mesh=mesh,
        out_type=jax.ShapeDtypeStruct((_N, 2 * _D), jnp.float32),
        scratch_types=[
            pltpu.VMEM((_B_PER_W,), jnp.int32),
            pltpu.VMEM((_CHUNK, 2 * _D), jnp.float32),
            pltpu.VMEM((_CHUNK, 2 * _D), jnp.float32),
            pltpu.SemaphoreType.DMA,
            pltpu.SemaphoreType.DMA,
        ],
    )
    def k(emb_hbm, idx_hbm, out_hbm, idx_v, buf0, buf1, sem0, sem1):
        wid = lax.axis_index("s") * _NC + lax.axis_index("c")
        base = wid * _B_PER_W
        pltpu.sync_copy(idx_hbm.at[pl.ds(base, _B_PER_W)], idx_v)
        bufs = (buf0, buf1)
        sems = (sem0, sem1)
        cps = []
        for c in range(_NCHUNK):
            b = c % 2
            cp = pltpu.make_async_copy(
                emb_hbm.at[idx_v.at[pl.ds(c * _CHUNK, _CHUNK)]], bufs[b], sems[b])
            cp.start()
            cps.append(cp)
            if c >= 1:
                cps[c - 1].wait()
                pltpu.sync_copy(
                    bufs[(c - 1) % 2],
                    out_hbm.at[pl.ds(base + (c - 1) * _CHUNK, _CHUNK)])
        cps[_NCHUNK - 1].wait()
        pltpu.sync_copy(
            bufs[(_NCHUNK - 1) % 2],
            out_hbm.at[pl.ds(base + (_NCHUNK - 1) * _CHUNK, _CHUNK)])

    return k(emb_lines, line_idx)


_BLK = 1024


def _mlp_body(lines_ref, par_ref, w1_ref, b1_ref, w2_ref, b2_ref, o_ref):
    lines = lines_ref[...]
    p0 = par_ref[:, 0:1] != 0
    p1 = par_ref[:, 1:2] != 0
    e1 = jnp.where(p0, lines[:, _D:2 * _D], lines[:, 0:_D])
    e2 = jnp.where(p1, lines[:, 3 * _D:4 * _D], lines[:, 2 * _D:3 * _D])
    cat = jnp.concatenate([e1, e2], axis=1)
    h = lax.dot_general(
        cat, w1_ref[...], (((1,), (1,)), ((), ())),
        preferred_element_type=jnp.float32,
        precision=lax.Precision.HIGHEST,
    )
    h = jnp.maximum(h + b1_ref[...], 0.0)
    o_ref[...] = lax.dot_general(
        h, w2_ref[...], (((1,), (1,)), ((), ())),
        preferred_element_type=jnp.float32,
        precision=lax.Precision.HIGHEST,
    ) + b2_ref[...]


def _mlp(lines, par, W1, b1, W2, b2):
    return pl.pallas_call(
        _mlp_body,
        grid=(_B // _BLK,),
        in_specs=[
            pl.BlockSpec((_BLK, 4 * _D), lambda i: (i, 0)),
            pl.BlockSpec((_BLK, 2), lambda i: (i, 0)),
            pl.BlockSpec((_H, 2 * _D), lambda i: (0, 0)),
            pl.BlockSpec((1, _H), lambda i: (0, 0)),
            pl.BlockSpec((_OUT, _H), lambda i: (0, 0)),
            pl.BlockSpec((1, _OUT), lambda i: (0, 0)),
        ],
        out_specs=pl.BlockSpec((_BLK, _OUT), lambda i: (i, 0)),
        out_shape=jax.ShapeDtypeStruct((_B, _OUT), jnp.float32),
    )(lines, par, W1, b1.reshape(1, _H), W2, b2.reshape(1, _OUT))


def kernel(x, emb, W1, b1, W2, b2):
    xi = x.astype(jnp.int32)
    idx = xi.reshape(_N)
    line_idx = idx >> 1
    par = xi & 1
    emb_lines = emb.reshape(_V // 2, 2 * _D)
    lines = _sc_gather_lines(emb_lines, line_idx)
    lines2 = lines.reshape(_B, 4 * _D)
    return _mlp(lines2, par, W1, b1, W2, b2)


# speedup vs baseline: 1.6867x; 1.6867x over previous
import functools

import jax
import jax.numpy as jnp
from jax import lax
from jax.experimental import pallas as pl
from jax.experimental.pallas import tpu as pltpu
from jax.experimental.pallas import tpu_sc as plsc

_V = 1000000
_D = 64
_H = 256
_OUT = 128
_B = 16384
_N = 2 * _B

_TBLK = 2048
_M = 512000
_NLINEBLK = _M // _TBLK
_LASTBLK = (_V + _TBLK - 1) // _TBLK - 1

_NC = 2
_NS = 16
_NW = _NC * _NS
_PER_W = _N // _NW
_CHUNK = 256
_NCHUNK = _PER_W // _CHUNK

_PARALLEL = pltpu.CompilerParams(dimension_semantics=("parallel",))
_ARB = pltpu.CompilerParams(dimension_semantics=("arbitrary",))


def _transpose_body(a_ref, b_ref, dst_ref):
    ta = jnp.transpose(a_ref[...], (1, 0))
    tb = jnp.transpose(b_ref[...], (1, 0))
    dst_ref[...] = jnp.concatenate([ta, tb], axis=1)


def _build_rowmajor(emb_t):
    return pl.pallas_call(
        _transpose_body,
        grid=(_NLINEBLK,),
        in_specs=[
            pl.BlockSpec((_D, _TBLK), lambda i: (0, i)),
            pl.BlockSpec(
                (_D, _TBLK),
                lambda i: (0, jnp.minimum(i + _NLINEBLK, _LASTBLK))),
        ],
        out_specs=pl.BlockSpec((_TBLK, 2 * _D), lambda i: (i, 0)),
        out_shape=jax.ShapeDtypeStruct((_M, 2 * _D), jnp.float32),
        compiler_params=_PARALLEL,
    )(emb_t, emb_t)


def _sc_gather(table, idx):
    mesh = plsc.VectorSubcoreMesh(core_axis_name="c", subcore_axis_name="s")

    @functools.partial(
        pl.kernel,
        mesh=mesh,
        out_type=jax.ShapeDtypeStruct((_N, 2 * _D), jnp.float32),
        scratch_types=[
            pltpu.VMEM((_PER_W,), jnp.int32),
            pltpu.VMEM((_CHUNK, 2 * _D), jnp.float32),
            pltpu.VMEM((_CHUNK, 2 * _D), jnp.float32),
            pltpu.SemaphoreType.DMA,
            pltpu.SemaphoreType.DMA,
        ],
    )
    def k(tab_hbm, idx_hbm, out_hbm, idx_v, buf0, buf1, sem0, sem1):
        wid = lax.axis_index("s") * _NC + lax.axis_index("c")
        base = wid * _PER_W
        pltpu.sync_copy(idx_hbm.at[pl.ds(base, _PER_W)], idx_v)
        bufs = (buf0, buf1)
        sems = (sem0, sem1)
        cps = []
        for c in range(_NCHUNK):
            b = c % 2
            cp = pltpu.make_async_copy(
                tab_hbm.at[idx_v.at[pl.ds(c * _CHUNK, _CHUNK)]], bufs[b], sems[b])
            cp.start()
            cps.append(cp)
            if c >= 1:
                cps[c - 1].wait()
                pltpu.sync_copy(
                    bufs[(c - 1) % 2],
                    out_hbm.at[pl.ds(base + (c - 1) * _CHUNK, _CHUNK)])
        cps[_NCHUNK - 1].wait()
        pltpu.sync_copy(
            bufs[(_NCHUNK - 1) % 2],
            out_hbm.at[pl.ds(base + (_NCHUNK - 1) * _CHUNK, _CHUNK)])

    return k(table, idx)


_BLK = 1024


def _mlp_body(g1_ref, g2_ref, par_ref, w1_ref, b1_ref, w2_ref, b2_ref, o_ref):
    p0 = par_ref[:, 0:1] != 0
    p1 = par_ref[:, 1:2] != 0
    e1 = jnp.where(p0, g1_ref[:, _D:2 * _D], g1_ref[:, 0:_D])
    e2 = jnp.where(p1, g2_ref[:, _D:2 * _D], g2_ref[:, 0:_D])
    cat = jnp.concatenate([e1, e2], axis=1)
    h = lax.dot_general(
        cat, w1_ref[...], (((1,), (1,)), ((), ())),
        preferred_element_type=jnp.float32,
        precision=lax.Precision.HIGHEST,
    )
    h = jnp.maximum(h + b1_ref[...], 0.0)
    o_ref[...] = lax.dot_general(
        h, w2_ref[...], (((1,), (1,)), ((), ())),
        preferred_element_type=jnp.float32,
        precision=lax.Precision.HIGHEST,
    ) + b2_ref[...]


def _mlp(rows, par, W1, b1, W2, b2):
    nblk = _B // _BLK
    return pl.pallas_call(
        _mlp_body,
        grid=(nblk,),
        in_specs=[
            pl.BlockSpec((_BLK, 2 * _D), lambda i: (i, 0)),
            pl.BlockSpec((_BLK, 2 * _D), lambda i: (i + nblk, 0)),
            pl.BlockSpec((_BLK, 2), lambda i: (i, 0)),
            pl.BlockSpec((_H, 2 * _D), lambda i: (0, 0)),
            pl.BlockSpec((1, _H), lambda i: (0, 0)),
            pl.BlockSpec((_OUT, _H), lambda i: (0, 0)),
            pl.BlockSpec((1, _OUT), lambda i: (0, 0)),
        ],
        out_specs=pl.BlockSpec((_BLK, _OUT), lambda i: (i, 0)),
        out_shape=jax.ShapeDtypeStruct((_B, _OUT), jnp.float32),
        compiler_params=_PARALLEL,
    )(rows, rows, par, W1, b1.reshape(1, _H), W2, b2.reshape(1, _OUT))


def kernel(x, emb, W1, b1, W2, b2):
    xi = x.astype(jnp.int32)
    idx = xi.T.reshape(_N)
    line_idx = jnp.where(idx < _M, idx, idx - _M)
    par = (xi >= _M).astype(jnp.int32)
    emb_t = emb.T
    table = _build_rowmajor(emb_t)
    rows = _sc_gather(table, line_idx)
    return _mlp(rows, par, W1, b1, W2, b2)


# speedup vs baseline: 2.9254x; 1.7344x over previous
import functools

import jax
import jax.numpy as jnp
from jax import lax
from jax.experimental import pallas as pl
from jax.experimental.pallas import tpu as pltpu
from jax.experimental.pallas import tpu_sc as plsc

_V = 1000000
_D = 64
_H = 256
_OUT = 128
_B = 16384
_N = 2 * _B

_TBLK = 4096
_M = 512000
_NLINEBLK = _M // _TBLK
_LASTBLK = (_V + _TBLK - 1) // _TBLK - 1

_NC = 2
_NS = 16
_NW = _NC * _NS
_PER_W = _N // _NW
_CHUNK = 256
_NCHUNK = _PER_W // _CHUNK

_PARALLEL = pltpu.CompilerParams(dimension_semantics=("parallel",))


def _transpose_body(a_ref, b_ref, dst_ref):
    stacked = jnp.concatenate([a_ref[...], b_ref[...]], axis=0)
    dst_ref[...] = jnp.transpose(stacked, (1, 0))


def _build_rowmajor(emb_t):
    return pl.pallas_call(
        _transpose_body,
        grid=(_NLINEBLK,),
        in_specs=[
            pl.BlockSpec((_D, _TBLK), lambda i: (0, i)),
            pl.BlockSpec(
                (_D, _TBLK),
                lambda i: (0, jnp.minimum(i + _NLINEBLK, _LASTBLK))),
        ],
        out_specs=pl.BlockSpec((_TBLK, 2 * _D), lambda i: (i, 0)),
        out_shape=jax.ShapeDtypeStruct((_M, 2 * _D), jnp.float32),
        compiler_params=_PARALLEL,
    )(emb_t, emb_t)


def _sc_gather(table, idx):
    mesh = plsc.VectorSubcoreMesh(core_axis_name="c", subcore_axis_name="s")

    @functools.partial(
        pl.kernel,
        mesh=mesh,
        out_type=jax.ShapeDtypeStruct((_N, 2 * _D), jnp.float32),
        scratch_types=[
            pltpu.VMEM((_PER_W,), jnp.int32),
            pltpu.VMEM((_CHUNK, 2 * _D), jnp.float32),
            pltpu.VMEM((_CHUNK, 2 * _D), jnp.float32),
            pltpu.SemaphoreType.DMA,
            pltpu.SemaphoreType.DMA,
        ],
    )
    def k(tab_hbm, idx_hbm, out_hbm, idx_v, buf0, buf1, sem0, sem1):
        wid = lax.axis_index("s") * _NC + lax.axis_index("c")
        base = wid * _PER_W
        pltpu.sync_copy(idx_hbm.at[pl.ds(base, _PER_W)], idx_v)
        bufs = (buf0, buf1)
        sems = (sem0, sem1)
        cps = []
        for c in range(_NCHUNK):
            b = c % 2
            cp = pltpu.make_async_copy(
                tab_hbm.at[idx_v.at[pl.ds(c * _CHUNK, _CHUNK)]], bufs[b], sems[b])
            cp.start()
            cps.append(cp)
            if c >= 1:
                cps[c - 1].wait()
                pltpu.sync_copy(
                    bufs[(c - 1) % 2],
                    out_hbm.at[pl.ds(base + (c - 1) * _CHUNK, _CHUNK)])
        cps[_NCHUNK - 1].wait()
        pltpu.sync_copy(
            bufs[(_NCHUNK - 1) % 2],
            out_hbm.at[pl.ds(base + (_NCHUNK - 1) * _CHUNK, _CHUNK)])

    return k(table, idx)


_BLK = 2048


def _mlp_body(g1_ref, g2_ref, par_ref, w1_ref, b1_ref, w2_ref, b2_ref, o_ref):
    p0 = par_ref[:, 0:1] != 0
    p1 = par_ref[:, 1:2] != 0
    e1 = jnp.where(p0, g1_ref[:, _D:2 * _D], g1_ref[:, 0:_D])
    e2 = jnp.where(p1, g2_ref[:, _D:2 * _D], g2_ref[:, 0:_D])
    cat = jnp.concatenate([e1, e2], axis=1)
    h = lax.dot_general(
        cat, w1_ref[...], (((1,), (1,)), ((), ())),
        preferred_element_type=jnp.float32,
    )
    h = jnp.maximum(h + b1_ref[...], 0.0)
    o_ref[...] = lax.dot_general(
        h, w2_ref[...], (((1,), (1,)), ((), ())),
        preferred_element_type=jnp.float32,
    ) + b2_ref[...]


def _mlp(rows, par, W1, b1, W2, b2):
    nblk = _B // _BLK
    return pl.pallas_call(
        _mlp_body,
        grid=(nblk,),
        in_specs=[
            pl.BlockSpec((_BLK, 2 * _D), lambda i: (i, 0)),
            pl.BlockSpec((_BLK, 2 * _D), lambda i: (i + nblk, 0)),
            pl.BlockSpec((_BLK, 2), lambda i: (i, 0)),
            pl.BlockSpec((_H, 2 * _D), lambda i: (0, 0)),
            pl.BlockSpec((1, _H), lambda i: (0, 0)),
            pl.BlockSpec((_OUT, _H), lambda i: (0, 0)),
            pl.BlockSpec((1, _OUT), lambda i: (0, 0)),
        ],
        out_specs=pl.BlockSpec((_BLK, _OUT), lambda i: (i, 0)),
        out_shape=jax.ShapeDtypeStruct((_B, _OUT), jnp.float32),
        compiler_params=_PARALLEL,
    )(rows, rows, par, W1, b1.reshape(1, _H), W2, b2.reshape(1, _OUT))


def kernel(x, emb, W1, b1, W2, b2):
    xi = x.astype(jnp.int32)
    idx = xi.T.reshape(_N)
    line_idx = jnp.where(idx < _M, idx, idx - _M)
    par = (xi >= _M).astype(jnp.int32)
    emb_t = emb.T
    table = _build_rowmajor(emb_t)
    rows = _sc_gather(table, line_idx)
    return _mlp(rows, par, W1, b1, W2, b2)


# speedup vs baseline: 3.3673x; 1.1510x over previous
import functools

import jax
import jax.numpy as jnp
from jax import lax
from jax.experimental import pallas as pl
from jax.experimental.pallas import tpu as pltpu
from jax.experimental.pallas import tpu_sc as plsc

_V = 1000000
_D = 64
_H = 256
_OUT = 128
_B = 16384
_N = 2 * _B

_TBLK = 10240
_M = 512000
_NLINEBLK = _M // _TBLK
_LASTBLK = (_V + _TBLK - 1) // _TBLK - 1

_NC = 2
_NS = 16
_NW = _NC * _NS
_PER_W = _N // _NW
_CHUNK = 256
_NCHUNK = _PER_W // _CHUNK

_PARALLEL = pltpu.CompilerParams(dimension_semantics=("parallel",))


def _transpose_body(a_ref, b_ref, dst_ref):
    stacked = jnp.concatenate([a_ref[...], b_ref[...]], axis=0)
    dst_ref[...] = jnp.transpose(stacked, (1, 0))


def _build_rowmajor(emb_t):
    return pl.pallas_call(
        _transpose_body,
        grid=(_NLINEBLK,),
        in_specs=[
            pl.BlockSpec((_D, _TBLK), lambda i: (0, i)),
            pl.BlockSpec(
                (_D, _TBLK),
                lambda i: (0, jnp.minimum(i + _NLINEBLK, _LASTBLK))),
        ],
        out_specs=pl.BlockSpec((_TBLK, 2 * _D), lambda i: (i, 0)),
        out_shape=jax.ShapeDtypeStruct((_M, 2 * _D), jnp.float32),
        compiler_params=_PARALLEL,
    )(emb_t, emb_t)


def _sc_gather(table, idx):
    mesh = plsc.VectorSubcoreMesh(core_axis_name="c", subcore_axis_name="s")

    @functools.partial(
        pl.kernel,
        mesh=mesh,
        out_type=jax.ShapeDtypeStruct((_N, 2 * _D), jnp.float32),
        scratch_types=[
            pltpu.VMEM((_PER_W,), jnp.int32),
            pltpu.VMEM((_CHUNK, 2 * _D), jnp.float32),
            pltpu.VMEM((_CHUNK, 2 * _D), jnp.float32),
            pltpu.SemaphoreType.DMA,
            pltpu.SemaphoreType.DMA,
        ],
    )
    def k(tab_hbm, idx_hbm, out_hbm, idx_v, buf0, buf1, sem0, sem1):
        wid = lax.axis_index("s") * _NC + lax.axis_index("c")
        base = wid * _PER_W
        pltpu.sync_copy(idx_hbm.at[pl.ds(base, _PER_W)], idx_v)
        bufs = (buf0, buf1)
        sems = (sem0, sem1)
        cps = []
        for c in range(_NCHUNK):
            b = c % 2
            cp = pltpu.make_async_copy(
                tab_hbm.at[idx_v.at[pl.ds(c * _CHUNK, _CHUNK)]], bufs[b], sems[b])
            cp.start()
            cps.append(cp)
            if c >= 1:
                cps[c - 1].wait()
                pltpu.sync_copy(
                    bufs[(c - 1) % 2],
                    out_hbm.at[pl.ds(base + (c - 1) * _CHUNK, _CHUNK)])
        cps[_NCHUNK - 1].wait()
        pltpu.sync_copy(
            bufs[(_NCHUNK - 1) % 2],
            out_hbm.at[pl.ds(base + (_NCHUNK - 1) * _CHUNK, _CHUNK)])

    return k(table, idx)


_BLK = 2048


def _mlp_body(g1_ref, g2_ref, par_ref, w1_ref, b1_ref, w2_ref, b2_ref, o_ref):
    p0 = par_ref[:, 0:1] != 0
    p1 = par_ref[:, 1:2] != 0
    e1 = jnp.where(p0, g1_ref[:, _D:2 * _D], g1_ref[:, 0:_D])
    e2 = jnp.where(p1, g2_ref[:, _D:2 * _D], g2_ref[:, 0:_D])
    cat = jnp.concatenate([e1, e2], axis=1)
    h = lax.dot_general(
        cat, w1_ref[...], (((1,), (1,)), ((), ())),
        preferred_element_type=jnp.float32,
    )
    h = jnp.maximum(h + b1_ref[...], 0.0)
    o_ref[...] = lax.dot_general(
        h, w2_ref[...], (((1,), (1,)), ((), ())),
        preferred_element_type=jnp.float32,
    ) + b2_ref[...]


def _mlp(rows, par, W1, b1, W2, b2):
    nblk = _B // _BLK
    return pl.pallas_call(
        _mlp_body,
        grid=(nblk,),
        in_specs=[
            pl.BlockSpec((_BLK, 2 * _D), lambda i: (i, 0)),
            pl.BlockSpec((_BLK, 2 * _D), lambda i: (i + nblk, 0)),
            pl.BlockSpec((_BLK, 2), lambda i: (i, 0)),
            pl.BlockSpec((_H, 2 * _D), lambda i: (0, 0)),
            pl.BlockSpec((1, _H), lambda i: (0, 0)),
            pl.BlockSpec((_OUT, _H), lambda i: (0, 0)),
            pl.BlockSpec((1, _OUT), lambda i: (0, 0)),
        ],
        out_specs=pl.BlockSpec((_BLK, _OUT), lambda i: (i, 0)),
        out_shape=jax.ShapeDtypeStruct((_B, _OUT), jnp.float32),
        compiler_params=_PARALLEL,
    )(rows, rows, par, W1, b1.reshape(1, _H), W2, b2.reshape(1, _OUT))


def kernel(x, emb, W1, b1, W2, b2):
    xi = x.astype(jnp.int32)
    idx = xi.T.reshape(_N)
    line_idx = jnp.where(idx < _M, idx, idx - _M)
    par = (xi >= _M).astype(jnp.int32)
    emb_t = emb.T
    table = _build_rowmajor(emb_t)
    rows = _sc_gather(table, line_idx)
    return _mlp(rows, par, W1, b1, W2, b2)


# speedup vs baseline: 3.4148x; 1.0141x over previous
import functools

import jax
import jax.numpy as jnp
from jax import lax
from jax.experimental import pallas as pl
from jax.experimental.pallas import tpu as pltpu
from jax.experimental.pallas import tpu_sc as plsc

_V = 1000000
_D = 64
_H = 256
_OUT = 128
_B = 16384
_N = 2 * _B

_TBLK = 20480
_M = 512000
_NLINEBLK = _M // _TBLK
_LASTBLK = (_V + _TBLK - 1) // _TBLK - 1

_NC = 2
_NS = 16
_NW = _NC * _NS
_PER_W = _N // _NW
_CHUNK = 256
_NCHUNK = _PER_W // _CHUNK

_PARALLEL = pltpu.CompilerParams(dimension_semantics=("parallel",))


def _transpose_body(a_ref, b_ref, dst_ref):
    stacked = jnp.concatenate([a_ref[...], b_ref[...]], axis=0)
    dst_ref[...] = jnp.transpose(stacked, (1, 0))


def _build_rowmajor(emb_t):
    return pl.pallas_call(
        _transpose_body,
        grid=(_NLINEBLK,),
        in_specs=[
            pl.BlockSpec((_D, _TBLK), lambda i: (0, i)),
            pl.BlockSpec(
                (_D, _TBLK),
                lambda i: (0, jnp.minimum(i + _NLINEBLK, _LASTBLK))),
        ],
        out_specs=pl.BlockSpec((_TBLK, 2 * _D), lambda i: (i, 0)),
        out_shape=jax.ShapeDtypeStruct((_M, 2 * _D), jnp.float32),
        compiler_params=_PARALLEL,
    )(emb_t, emb_t)


def _sc_gather(table, idx):
    mesh = plsc.VectorSubcoreMesh(core_axis_name="c", subcore_axis_name="s")

    @functools.partial(
        pl.kernel,
        mesh=mesh,
        out_type=jax.ShapeDtypeStruct((_N, 2 * _D), jnp.float32),
        scratch_types=[
            pltpu.VMEM((_PER_W,), jnp.int32),
            pltpu.VMEM((_CHUNK, 2 * _D), jnp.float32),
            pltpu.VMEM((_CHUNK, 2 * _D), jnp.float32),
            pltpu.SemaphoreType.DMA,
            pltpu.SemaphoreType.DMA,
        ],
    )
    def k(tab_hbm, idx_hbm, out_hbm, idx_v, buf0, buf1, sem0, sem1):
        wid = lax.axis_index("s") * _NC + lax.axis_index("c")
        base = wid * _PER_W
        pltpu.sync_copy(idx_hbm.at[pl.ds(base, _PER_W)], idx_v)
        bufs = (buf0, buf1)
        sems = (sem0, sem1)
        cps = []
        for c in range(_NCHUNK):
            b = c % 2
            cp = pltpu.make_async_copy(
                tab_hbm.at[idx_v.at[pl.ds(c * _CHUNK, _CHUNK)]], bufs[b], sems[b])
            cp.start()
            cps.append(cp)
            if c >= 1:
                cps[c - 1].wait()
                pltpu.sync_copy(
                    bufs[(c - 1) % 2],
                    out_hbm.at[pl.ds(base + (c - 1) * _CHUNK, _CHUNK)])
        cps[_NCHUNK - 1].wait()
        pltpu.sync_copy(
            bufs[(_NCHUNK - 1) % 2],
            out_hbm.at[pl.ds(base + (_NCHUNK - 1) * _CHUNK, _CHUNK)])

    return k(table, idx)


_BLK = 2048


def _mlp_body(g1_ref, g2_ref, par_ref, w1_ref, b1_ref, w2_ref, b2_ref, o_ref):
    p0 = par_ref[:, 0:1] != 0
    p1 = par_ref[:, 1:2] != 0
    e1 = jnp.where(p0, g1_ref[:, _D:2 * _D], g1_ref[:, 0:_D])
    e2 = jnp.where(p1, g2_ref[:, _D:2 * _D], g2_ref[:, 0:_D])
    cat = jnp.concatenate([e1, e2], axis=1)
    h = lax.dot_general(
        cat, w1_ref[...], (((1,), (1,)), ((), ())),
        preferred_element_type=jnp.float32,
    )
    h = jnp.maximum(h + b1_ref[...], 0.0)
    o_ref[...] = lax.dot_general(
        h, w2_ref[...], (((1,), (1,)), ((), ())),
        preferred_element_type=jnp.float32,
    ) + b2_ref[...]


def _mlp(rows, par, W1, b1, W2, b2):
    nblk = _B // _BLK
    return pl.pallas_call(
        _mlp_body,
        grid=(nblk,),
        in_specs=[
            pl.BlockSpec((_BLK, 2 * _D), lambda i: (i, 0)),
            pl.BlockSpec((_BLK, 2 * _D), lambda i: (i + nblk, 0)),
            pl.BlockSpec((_BLK, 2), lambda i: (i, 0)),
            pl.BlockSpec((_H, 2 * _D), lambda i: (0, 0)),
            pl.BlockSpec((1, _H), lambda i: (0, 0)),
            pl.BlockSpec((_OUT, _H), lambda i: (0, 0)),
            pl.BlockSpec((1, _OUT), lambda i: (0, 0)),
        ],
        out_specs=pl.BlockSpec((_BLK, _OUT), lambda i: (i, 0)),
        out_shape=jax.ShapeDtypeStruct((_B, _OUT), jnp.float32),
        compiler_params=_PARALLEL,
    )(rows, rows, par, W1, b1.reshape(1, _H), W2, b2.reshape(1, _OUT))


def kernel(x, emb, W1, b1, W2, b2):
    xi = x.astype(jnp.int32)
    idx = xi.T.reshape(_N)
    line_idx = jnp.where(idx < _M, idx, idx - _M)
    par = (xi >= _M).astype(jnp.int32)
    emb_t = emb.T
    table = _build_rowmajor(emb_t)
    rows = _sc_gather(table, line_idx)
    return _mlp(rows, par, W1, b1, W2, b2)
